# Initial kernel scaffold; baseline (speedup 1.0000x reference)
import jax
import jax.numpy as jnp
from jax.experimental import pallas as pl


def kernel(Z, embeddings):
    raise NotImplementedError("write your pallas kernel here")


# speedup vs baseline: 3.9078x; 3.9078x over previous
import functools

import jax
import jax.numpy as jnp
from jax import lax
from jax.experimental import pallas as pl
from jax.experimental.pallas import tpu as pltpu
from jax.experimental.pallas import tpu_sc as plsc

N_ATOMS = 1048576
EMB = 128
NUM_CORES = 2
NUM_SUBCORES = 16
NUM_WORKERS = NUM_CORES * NUM_SUBCORES
B_PER_W = N_ATOMS // NUM_WORKERS
IDX_MINOR = 128
IDX_ROWS = 8
CHUNK = IDX_MINOR * IDX_ROWS
HALF = CHUNK // 2
N_CHUNKS = B_PER_W // CHUNK

_mesh = plsc.VectorSubcoreMesh(core_axis_name="c", subcore_axis_name="s")


@functools.partial(
    pl.kernel,
    out_type=jax.ShapeDtypeStruct((N_ATOMS, EMB), jnp.float32),
    mesh=_mesh,
    scratch_types=[
        pltpu.VMEM((IDX_ROWS, IDX_MINOR), jnp.int32),
        pltpu.VMEM((HALF, EMB), jnp.float32),
        pltpu.SemaphoreType.DMA,
    ],
)
def _emb_lookup(table_hbm, z2d_hbm, out_hbm, idx_v, rows_v, sem):
    wid = lax.axis_index("s") * NUM_CORES + lax.axis_index("c")
    base = wid * B_PER_W

    def body(i, carry):
        off = base + i * CHUNK
        idx_row = pl.multiple_of(off // IDX_MINOR, IDX_ROWS)
        pltpu.sync_copy(z2d_hbm.at[pl.ds(idx_row, IDX_ROWS)], idx_v)
        for h in range(2):
            copies = []
            for j in range(IDX_ROWS // 2):
                copies.append(
                    pltpu.async_copy(
                        table_hbm.at[idx_v.at[h * (IDX_ROWS // 2) + j]],
                        rows_v.at[pl.ds(j * IDX_MINOR, IDX_MINOR)],
                        sem,
                    )
                )
            for c in copies:
                c.wait()
            pltpu.sync_copy(rows_v, out_hbm.at[pl.ds(off + h * HALF, HALF)])
        return carry

    lax.fori_loop(0, N_CHUNKS, body, 0)


def kernel(Z, embeddings):
    table = jnp.concatenate(
        [jnp.zeros((1, EMB), embeddings.dtype), embeddings], axis=0
    )
    z2d = Z.astype(jnp.int32).reshape(N_ATOMS // IDX_MINOR, IDX_MINOR)
    return _emb_lookup(table, z2d)


# speedup vs baseline: 12.2819x; 3.1429x over previous
import functools

import jax
import jax.numpy as jnp
from jax import lax
from jax.experimental import pallas as pl
from jax.experimental.pallas import tpu as pltpu
from jax.experimental.pallas import tpu_sc as plsc

N_ATOMS = 1048576
EMB = 128
NUM_CORES = 2
NUM_SUBCORES = 16
NUM_WORKERS = NUM_CORES * NUM_SUBCORES
B_PER_W = N_ATOMS // NUM_WORKERS
IDX_MINOR = 128
IDX_ROWS = 8
CHUNK = IDX_MINOR * IDX_ROWS
HALF = CHUNK // 2
N_CHUNKS = B_PER_W // CHUNK
TAB_ROWS = 104

_mesh = plsc.VectorSubcoreMesh(core_axis_name="c", subcore_axis_name="s")


@functools.partial(
    pl.kernel,
    out_type=jax.ShapeDtypeStruct((N_ATOMS, EMB), jnp.float32),
    mesh=_mesh,
    scratch_types=[
        pltpu.VMEM((IDX_ROWS, IDX_MINOR), jnp.int32),
        pltpu.VMEM((HALF, EMB), jnp.float32),
        pltpu.VMEM_SHARED((TAB_ROWS, EMB), jnp.float32),
        pltpu.SemaphoreType.DMA,
    ],
)
def _emb_lookup(table_hbm, z2d_hbm, out_hbm, idx_v, rows_v, table_sh, sem):
    wid = lax.axis_index("s") * NUM_CORES + lax.axis_index("c")
    base = wid * B_PER_W

    @pl.when(lax.axis_index("s") == 0)
    def _():
        pltpu.sync_copy(table_hbm, table_sh)

    plsc.subcore_barrier()

    def body(i, carry):
        off = base + i * CHUNK
        idx_row = pl.multiple_of(off // IDX_MINOR, IDX_ROWS)
        pltpu.sync_copy(z2d_hbm.at[pl.ds(idx_row, IDX_ROWS)], idx_v)
        for h in range(2):
            copies = []
            for j in range(IDX_ROWS // 2):
                copies.append(
                    pltpu.async_copy(
                        table_sh.at[idx_v.at[h * (IDX_ROWS // 2) + j]],
                        rows_v.at[pl.ds(j * IDX_MINOR, IDX_MINOR)],
                        sem,
                    )
                )
            for c in copies:
                c.wait()
            pltpu.sync_copy(rows_v, out_hbm.at[pl.ds(off + h * HALF, HALF)])
        return carry

    lax.fori_loop(0, N_CHUNKS, body, 0)


def kernel(Z, embeddings):
    table = jnp.concatenate(
        [
            jnp.zeros((1, EMB), embeddings.dtype),
            embeddings,
            jnp.zeros((TAB_ROWS - 1 - embeddings.shape[0], EMB), embeddings.dtype),
        ],
        axis=0,
    )
    z2d = Z.astype(jnp.int32).reshape(N_ATOMS // IDX_MINOR, IDX_MINOR)
    return _emb_lookup(table, z2d)


# speedup vs baseline: 19.4525x; 1.5838x over previous
import functools

import jax
import jax.numpy as jnp
from jax import lax
from jax.experimental import pallas as pl
from jax.experimental.pallas import tpu as pltpu
from jax.experimental.pallas import tpu_sc as plsc

N_ATOMS = 1048576
EMB = 128
NUM_CORES = 2
NUM_SUBCORES = 16
NUM_WORKERS = NUM_CORES * NUM_SUBCORES
B_PER_W = N_ATOMS // NUM_WORKERS
IDX_MINOR = 128
IDX_ROWS = 8
CHUNK = IDX_MINOR * IDX_ROWS
N_CHUNKS = B_PER_W // CHUNK
BUF_ROWS = 256
GPB = BUF_ROWS // IDX_MINOR
UNITS = CHUNK // BUF_ROWS
TAB_ROWS = 104

_mesh = plsc.VectorSubcoreMesh(core_axis_name="c", subcore_axis_name="s")


@functools.partial(
    pl.kernel,
    out_type=jax.ShapeDtypeStruct((N_ATOMS, EMB), jnp.float32),
    mesh=_mesh,
    scratch_types=[
        pltpu.VMEM((2, IDX_ROWS, IDX_MINOR), jnp.int32),
        pltpu.VMEM((2, BUF_ROWS, EMB), jnp.float32),
        pltpu.VMEM_SHARED((TAB_ROWS, EMB), jnp.float32),
        pltpu.SemaphoreType.DMA,
        pltpu.SemaphoreType.DMA,
        pltpu.SemaphoreType.DMA,
        pltpu.SemaphoreType.DMA,
        pltpu.SemaphoreType.DMA,
    ],
)
def _emb_lookup(
    table_hbm, z2d_hbm, out_hbm,
    idx_v, rows_v, table_sh, g_sem, i_sem0, i_sem1, st_sem0, st_sem1,
):
    wid = lax.axis_index("s") * NUM_CORES + lax.axis_index("c")
    base = wid * B_PER_W
    idx_base = base // IDX_MINOR
    i_sems = (i_sem0, i_sem1)
    st_sems = (st_sem0, st_sem1)

    @pl.when(lax.axis_index("s") == 0)
    def _():
        pltpu.sync_copy(table_hbm, table_sh)

    plsc.subcore_barrier()

    def fire_idx(chunk, d):
        row = pl.multiple_of(idx_base + chunk * IDX_ROWS, IDX_ROWS)
        pltpu.async_copy(
            z2d_hbm.at[pl.ds(row, IDX_ROWS)], idx_v.at[d], i_sems[d]
        )

    def wait_store(b):
        pltpu.make_async_copy(
            rows_v.at[b], out_hbm.at[pl.ds(0, BUF_ROWS)], st_sems[b]
        ).wait()

    def do_chunk(chunk, d, not_first):
        pltpu.make_async_copy(
            z2d_hbm.at[pl.ds(0, IDX_ROWS)], idx_v.at[d], i_sems[d]
        ).wait()
        off = base + chunk * CHUNK
        for u in range(UNITS):
            b = u % 2
            if u >= 2 or not_first is None:
                wait_store(b)
            else:
                @pl.when(not_first)
                def _():
                    wait_store(b)
            copies = []
            for g in range(GPB):
                copies.append(
                    pltpu.async_copy(
                        table_sh.at[idx_v.at[d, u * GPB + g]],
                        rows_v.at[b, pl.ds(g * IDX_MINOR, IDX_MINOR)],
                        g_sem,
                    )
                )
            for c in copies:
                c.wait()
            pltpu.async_copy(
                rows_v.at[b],
                out_hbm.at[pl.ds(off + u * BUF_ROWS, BUF_ROWS)],
                st_sems[b],
            )

    fire_idx(0, 0)
    fire_idx(1, 1)

    def body(s, carry):
        c0 = 2 * s
        do_chunk(c0, 0, not_first=(s > 0))
        @pl.when(s < N_CHUNKS // 2 - 1)
        def _():
            fire_idx(c0 + 2, 0)
        do_chunk(c0 + 1, 1, not_first=None)
        @pl.when(s < N_CHUNKS // 2 - 1)
        def _():
            fire_idx(c0 + 3, 1)
        return carry

    lax.fori_loop(0, N_CHUNKS // 2, body, 0)
    wait_store(0)
    wait_store(1)


def kernel(Z, embeddings):
    table = jnp.concatenate(
        [
            jnp.zeros((1, EMB), embeddings.dtype),
            embeddings,
            jnp.zeros((TAB_ROWS - 1 - embeddings.shape[0], EMB), embeddings.dtype),
        ],
        axis=0,
    )
    z2d = Z.astype(jnp.int32).reshape(N_ATOMS // IDX_MINOR, IDX_MINOR)
    return _emb_lookup(table, z2d)


# speedup vs baseline: 19.9219x; 1.0241x over previous
import functools

import jax
import jax.numpy as jnp
from jax import lax
from jax.experimental import pallas as pl
from jax.experimental.pallas import tpu as pltpu
from jax.experimental.pallas import tpu_sc as plsc

N_ATOMS = 1048576
EMB = 128
NUM_CORES = 2
NUM_SUBCORES = 16
NUM_WORKERS = NUM_CORES * NUM_SUBCORES
B_PER_W = N_ATOMS // NUM_WORKERS
IDX_MINOR = 128
IDX_ROWS = B_PER_W // IDX_MINOR
BUF_ROWS = 128
NBUF = 4
N_UNITS = B_PER_W // BUF_ROWS
TAB_ROWS = 104

_mesh = plsc.VectorSubcoreMesh(core_axis_name="c", subcore_axis_name="s")


@functools.partial(
    pl.kernel,
    out_type=jax.ShapeDtypeStruct((N_ATOMS, EMB), jnp.float32),
    mesh=_mesh,
    scratch_types=[
        pltpu.VMEM((IDX_ROWS, IDX_MINOR), jnp.int32),
        pltpu.VMEM((NBUF, BUF_ROWS, EMB), jnp.float32),
        pltpu.VMEM_SHARED((TAB_ROWS, EMB), jnp.float32),
        pltpu.SemaphoreType.DMA,
        pltpu.SemaphoreType.DMA,
        pltpu.SemaphoreType.DMA,
        pltpu.SemaphoreType.DMA,
        pltpu.SemaphoreType.DMA,
        pltpu.SemaphoreType.DMA,
        pltpu.SemaphoreType.DMA,
        pltpu.SemaphoreType.DMA,
        pltpu.SemaphoreType.DMA,
    ],
)
def _emb_lookup(
    table_hbm, z2d_hbm, out_hbm,
    idx_v, rows_v, table_sh, i_sem,
    g_sem0, g_sem1, g_sem2, g_sem3,
    st_sem0, st_sem1, st_sem2, st_sem3,
):
    wid = lax.axis_index("s") * NUM_CORES + lax.axis_index("c")
    base = wid * B_PER_W
    g_sems = (g_sem0, g_sem1, g_sem2, g_sem3)
    st_sems = (st_sem0, st_sem1, st_sem2, st_sem3)

    idx_row = pl.multiple_of(wid * IDX_ROWS, IDX_ROWS)
    idx_cp = pltpu.async_copy(
        z2d_hbm.at[pl.ds(idx_row, IDX_ROWS)], idx_v, i_sem
    )

    @pl.when(lax.axis_index("s") == 0)
    def _():
        pltpu.sync_copy(table_hbm, table_sh)

    plsc.subcore_barrier()
    idx_cp.wait()

    def issue_gather(u, b):
        pltpu.async_copy(
            table_sh.at[idx_v.at[u]], rows_v.at[b], g_sems[b]
        )

    def wait_gather(b):
        pltpu.make_async_copy(
            table_sh.at[idx_v.at[0]], rows_v.at[b], g_sems[b]
        ).wait()

    def issue_store(u, b):
        pltpu.async_copy(
            rows_v.at[b],
            out_hbm.at[pl.ds(base + u * BUF_ROWS, BUF_ROWS)],
            st_sems[b],
        )

    def wait_store(b):
        pltpu.make_async_copy(
            rows_v.at[b], out_hbm.at[pl.ds(0, BUF_ROWS)], st_sems[b]
        ).wait()

    def body(s, carry):
        for j in range(NBUF):
            u = NBUF * s + j
            b = j
            pb = (j - 1) % NBUF
            if j == 0:
                @pl.when(s > 0)
                def _():
                    wait_store(b)
                    issue_gather(u, b)
                    wait_gather(pb)
                    issue_store(u - 1, pb)
                @pl.when(s == 0)
                def _():
                    issue_gather(u, b)
            else:
                @pl.when(s > 0)
                def _():
                    wait_store(b)
                issue_gather(u, b)
                wait_gather(pb)
                issue_store(u - 1, pb)
        return carry

    lax.fori_loop(0, N_UNITS // NBUF, body, 0)
    wait_gather(NBUF - 1)
    issue_store(N_UNITS - 1, NBUF - 1)
    for b in range(NBUF):
        wait_store(b)


def kernel(Z, embeddings):
    table = jnp.concatenate(
        [
            jnp.zeros((1, EMB), embeddings.dtype),
            embeddings,
            jnp.zeros((TAB_ROWS - 1 - embeddings.shape[0], EMB), embeddings.dtype),
        ],
        axis=0,
    )
    z2d = Z.astype(jnp.int32).reshape(N_ATOMS // IDX_MINOR, IDX_MINOR)
    return _emb_lookup(table, z2d)


# speedup vs baseline: 20.1706x; 1.0125x over previous
import functools

import jax
import jax.numpy as jnp
from jax import lax
from jax.experimental import pallas as pl
from jax.experimental.pallas import tpu as pltpu
from jax.experimental.pallas import tpu_sc as plsc

N_ATOMS = 1048576
EMB = 128
NUM_CORES = 2
NUM_SUBCORES = 16
NUM_WORKERS = NUM_CORES * NUM_SUBCORES
B_PER_W = N_ATOMS // NUM_WORKERS
IDX_MINOR = 128
IDX_ROWS = B_PER_W // IDX_MINOR
BUF_ROWS = 64
NBUF = 8
N_UNITS = B_PER_W // BUF_ROWS
TAB_ROWS = 104

_mesh = plsc.VectorSubcoreMesh(core_axis_name="c", subcore_axis_name="s")


@functools.partial(
    pl.kernel,
    out_type=jax.ShapeDtypeStruct((N_ATOMS, EMB), jnp.float32),
    mesh=_mesh,
    scratch_types=[
        pltpu.VMEM((IDX_ROWS, IDX_MINOR), jnp.int32),
        pltpu.VMEM((NBUF, BUF_ROWS, EMB), jnp.float32),
        pltpu.VMEM_SHARED((TAB_ROWS, EMB), jnp.float32),
        pltpu.SemaphoreType.DMA,
        pltpu.SemaphoreType.DMA,
        pltpu.SemaphoreType.DMA,
        pltpu.SemaphoreType.DMA,
        pltpu.SemaphoreType.DMA,
        pltpu.SemaphoreType.DMA,
        pltpu.SemaphoreType.DMA,
        pltpu.SemaphoreType.DMA,
        pltpu.SemaphoreType.DMA,
        pltpu.SemaphoreType.DMA,
        pltpu.SemaphoreType.DMA,
        pltpu.SemaphoreType.DMA,
        pltpu.SemaphoreType.DMA,
        pltpu.SemaphoreType.DMA,
        pltpu.SemaphoreType.DMA,
        pltpu.SemaphoreType.DMA,
        pltpu.SemaphoreType.DMA,

    ],
)
def _emb_lookup(
    table_hbm, z2d_hbm, out_hbm,
    idx_v, rows_v, table_sh, i_sem,
    g_sem0, g_sem1, g_sem2, g_sem3, g_sem4, g_sem5, g_sem6, g_sem7,
    st_sem0, st_sem1, st_sem2, st_sem3, st_sem4, st_sem5, st_sem6, st_sem7,
):
    wid = lax.axis_index("s") * NUM_CORES + lax.axis_index("c")
    base = wid * B_PER_W
    g_sems = (g_sem0, g_sem1, g_sem2, g_sem3, g_sem4, g_sem5, g_sem6, g_sem7)
    st_sems = (st_sem0, st_sem1, st_sem2, st_sem3, st_sem4, st_sem5, st_sem6, st_sem7)

    idx_row = pl.multiple_of(wid * IDX_ROWS, IDX_ROWS)
    idx_cp = pltpu.async_copy(
        z2d_hbm.at[pl.ds(idx_row, IDX_ROWS)], idx_v, i_sem
    )

    @pl.when(lax.axis_index("s") == 0)
    def _():
        pltpu.sync_copy(table_hbm, table_sh)

    plsc.subcore_barrier()
    idx_cp.wait()

    def issue_gather(u, b):
        pltpu.async_copy(
            table_sh.at[idx_v.at[u // 2, pl.ds((u % 2) * BUF_ROWS, BUF_ROWS)]],
            rows_v.at[b],
            g_sems[b],
        )

    def wait_gather(b):
        pltpu.make_async_copy(
            table_sh.at[idx_v.at[0, pl.ds(0, BUF_ROWS)]], rows_v.at[b], g_sems[b]
        ).wait()

    def issue_store(u, b):
        pltpu.async_copy(
            rows_v.at[b],
            out_hbm.at[pl.ds(base + u * BUF_ROWS, BUF_ROWS)],
            st_sems[b],
        )

    def wait_store(b):
        pltpu.make_async_copy(
            rows_v.at[b], out_hbm.at[pl.ds(0, BUF_ROWS)], st_sems[b]
        ).wait()

    def body(s, carry):
        for j in range(NBUF):
            u = NBUF * s + j
            b = j
            pb = (j - 1) % NBUF
            if j == 0:
                @pl.when(s > 0)
                def _():
                    wait_store(b)
                    issue_gather(u, b)
                    wait_gather(pb)
                    issue_store(u - 1, pb)
                @pl.when(s == 0)
                def _():
                    issue_gather(u, b)
            else:
                @pl.when(s > 0)
                def _():
                    wait_store(b)
                issue_gather(u, b)
                wait_gather(pb)
                issue_store(u - 1, pb)
        return carry

    lax.fori_loop(0, N_UNITS // NBUF, body, 0)
    wait_gather(NBUF - 1)
    issue_store(N_UNITS - 1, NBUF - 1)
    for b in range(NBUF):
        wait_store(b)


def kernel(Z, embeddings):
    table = jnp.concatenate(
        [
            jnp.zeros((1, EMB), embeddings.dtype),
            embeddings,
            jnp.zeros((TAB_ROWS - 1 - embeddings.shape[0], EMB), embeddings.dtype),
        ],
        axis=0,
    )
    z2d = Z.astype(jnp.int32).reshape(N_ATOMS // IDX_MINOR, IDX_MINOR)
    return _emb_lookup(table, z2d)


# speedup vs baseline: 20.2096x; 1.0019x over previous
import functools

import jax
import jax.numpy as jnp
from jax import lax
from jax.experimental import pallas as pl
from jax.experimental.pallas import tpu as pltpu
from jax.experimental.pallas import tpu_sc as plsc

N_ATOMS = 1048576
EMB = 128
NUM_CORES = 2
NUM_SUBCORES = 16
NUM_WORKERS = NUM_CORES * NUM_SUBCORES
B_PER_W = N_ATOMS // NUM_WORKERS
IDX_MINOR = 128
IDX_ROWS = B_PER_W // IDX_MINOR
BUF_ROWS = 64
NBUF = 8
N_UNITS = B_PER_W // BUF_ROWS
TAB_ROWS = 104

_mesh = plsc.VectorSubcoreMesh(core_axis_name="c", subcore_axis_name="s")


@functools.partial(
    pl.kernel,
    out_type=jax.ShapeDtypeStruct((N_ATOMS, EMB), jnp.float32),
    mesh=_mesh,
    scratch_types=[
        pltpu.VMEM((IDX_ROWS, IDX_MINOR), jnp.int32),
        pltpu.VMEM((NBUF, BUF_ROWS, EMB), jnp.float32),
        pltpu.VMEM_SHARED((TAB_ROWS, EMB), jnp.float32),
        pltpu.SemaphoreType.DMA,
        pltpu.SemaphoreType.DMA,
        pltpu.SemaphoreType.DMA,
        pltpu.SemaphoreType.DMA,
        pltpu.SemaphoreType.DMA,
        pltpu.SemaphoreType.DMA,
        pltpu.SemaphoreType.DMA,
        pltpu.SemaphoreType.DMA,
        pltpu.SemaphoreType.DMA,
        pltpu.SemaphoreType.DMA,
        pltpu.SemaphoreType.DMA,
        pltpu.SemaphoreType.DMA,
        pltpu.SemaphoreType.DMA,
        pltpu.SemaphoreType.DMA,
        pltpu.SemaphoreType.DMA,
        pltpu.SemaphoreType.DMA,
        pltpu.SemaphoreType.DMA,

    ],
)
def _emb_lookup(
    table_hbm, z2d_hbm, out_hbm,
    idx_v, rows_v, table_sh, i_sem,
    g_sem0, g_sem1, g_sem2, g_sem3, g_sem4, g_sem5, g_sem6, g_sem7,
    st_sem0, st_sem1, st_sem2, st_sem3, st_sem4, st_sem5, st_sem6, st_sem7,
):
    wid = lax.axis_index("s") * NUM_CORES + lax.axis_index("c")
    base = wid * B_PER_W
    g_sems = (g_sem0, g_sem1, g_sem2, g_sem3, g_sem4, g_sem5, g_sem6, g_sem7)
    st_sems = (st_sem0, st_sem1, st_sem2, st_sem3, st_sem4, st_sem5, st_sem6, st_sem7)

    idx_row = pl.multiple_of(wid * IDX_ROWS, IDX_ROWS)
    idx_cp = pltpu.async_copy(
        z2d_hbm.at[pl.ds(idx_row, IDX_ROWS)], idx_v, i_sem
    )

    @pl.when(lax.axis_index("s") == 0)
    def _():
        pltpu.sync_copy(table_hbm, table_sh)

    plsc.subcore_barrier()
    idx_cp.wait()

    def issue_gather(u, b):
        pltpu.async_copy(
            table_sh.at[idx_v.at[u // 2, pl.ds((u % 2) * BUF_ROWS, BUF_ROWS)]],
            rows_v.at[b],
            g_sems[b],
        )

    def wait_gather(b):
        pltpu.make_async_copy(
            table_sh.at[idx_v.at[0, pl.ds(0, BUF_ROWS)]], rows_v.at[b], g_sems[b]
        ).wait()

    def issue_store(u, b):
        pltpu.async_copy(
            rows_v.at[b],
            out_hbm.at[pl.ds(base + u * BUF_ROWS, BUF_ROWS)],
            st_sems[b],
        )

    def wait_store(b):
        pltpu.make_async_copy(
            rows_v.at[b], out_hbm.at[pl.ds(0, BUF_ROWS)], st_sems[b]
        ).wait()

    def body(s, carry):
        for j in range(NBUF):
            u = NBUF * s + j
            b = j
            pb = (j - 2) % NBUF
            if j < 2:
                @pl.when(s > 0)
                def _():
                    wait_store(b)
                    issue_gather(u, b)
                    wait_gather(pb)
                    issue_store(u - 2, pb)
                @pl.when(s == 0)
                def _():
                    issue_gather(u, b)
            else:
                @pl.when(s > 0)
                def _():
                    wait_store(b)
                issue_gather(u, b)
                wait_gather(pb)
                issue_store(u - 2, pb)
        return carry

    lax.fori_loop(0, N_UNITS // NBUF, body, 0)
    for k in (2, 1):
        wait_gather(NBUF - k)
        issue_store(N_UNITS - k, NBUF - k)
    for b in range(NBUF):
        wait_store(b)


def kernel(Z, embeddings):
    table = jnp.concatenate(
        [
            jnp.zeros((1, EMB), embeddings.dtype),
            embeddings,
            jnp.zeros((TAB_ROWS - 1 - embeddings.shape[0], EMB), embeddings.dtype),
        ],
        axis=0,
    )
    z2d = Z.astype(jnp.int32).reshape(N_ATOMS // IDX_MINOR, IDX_MINOR)
    return _emb_lookup(table, z2d)
